# baseline (speedup 1.0000x reference)
import functools

import jax
import jax.numpy as jnp
from jax import lax
from jax.experimental import pallas as pl
from jax.experimental.pallas import tpu as pltpu
from jax.experimental.pallas import tpu_sc as plsc

_N = 10000
_E = 320000
_D = 128
_G = 64
_C = 10

_NC = 2
_NS = 16
_EPC = _E // _NC
_EPS = _EPC // _NS
_CHUNK = 64
_EPSP = 10240
_PAD = _EPSP - _EPS
_IB = 10
_NGRP = _EPSP // (_CHUNK * _IB)
_AROWS = _N + 16
_RPS = 624
_ZROWS = 16


def _seg_sum(h, pk):
  mesh = plsc.VectorSubcoreMesh(core_axis_name="c", subcore_axis_name="s")

  @functools.partial(
      pl.kernel,
      out_type=jax.ShapeDtypeStruct((_NC, _N, _D), jnp.float32),
      mesh=mesh,
      scratch_types=[
          pltpu.VMEM((_IB, 2, _CHUNK), jnp.int32),
          pltpu.VMEM((_CHUNK, _D), jnp.float32),
          pltpu.VMEM((_CHUNK, _D), jnp.float32),
          pltpu.VMEM((_ZROWS, _D), jnp.float32),
          pltpu.VMEM_SHARED((_AROWS, _D), jnp.float32),
          pltpu.SemaphoreType.DMA,
          pltpu.SemaphoreType.DMA,
      ],
  )
  def k(h_hbm, pk_hbm, out_hbm,
        ib, rows0, rows1, zero_v, acc_sh, sem0, sem1):
    cid = lax.axis_index("c")
    sid = lax.axis_index("s")
    my_pk = pk_hbm.at[cid].at[sid]

    @pl.loop(0, _ZROWS)
    def _(r):
      @pl.loop(0, _D, step=16)
      def _(c0):
        zero_v[r, pl.ds(c0, 16)] = jnp.zeros((16,), jnp.float32)

    base = sid * _RPS

    @pl.loop(0, _RPS // _ZROWS)
    def _(i):
      pltpu.sync_copy(zero_v, acc_sh.at[pl.ds(base + i * _ZROWS, _ZROWS)])

    @pl.when(sid == _NS - 1)
    def _():
      pltpu.sync_copy(zero_v, acc_sh.at[pl.ds(base + _RPS, _ZROWS)])

    plsc.subcore_barrier()

    @pl.loop(0, _NGRP)
    def _(g):
      pltpu.sync_copy(my_pk.at[g], ib)
      pltpu.async_copy(h_hbm.at[ib.at[0].at[0]], rows0, sem0)

      @pl.loop(0, _IB, step=2)
      def _(t):
        pltpu.async_copy(h_hbm.at[ib.at[t + 1].at[0]], rows1, sem1)
        pltpu.make_async_copy(h_hbm.at[ib.at[t].at[0]], rows0, sem0).wait()
        pltpu.sync_copy(rows0, acc_sh.at[ib.at[t].at[1]], add=True)

        @pl.when(t < _IB - 2)
        def _():
          pltpu.async_copy(h_hbm.at[ib.at[t + 2].at[0]], rows0, sem0)

        pltpu.make_async_copy(h_hbm.at[ib.at[t + 1].at[0]], rows1, sem1).wait()
        pltpu.sync_copy(rows1, acc_sh.at[ib.at[t + 1].at[1]], add=True)

    plsc.subcore_barrier()

    my_out = out_hbm.at[cid]

    @pl.when(sid < _NS - 1)
    def _():
      pltpu.sync_copy(acc_sh.at[pl.ds(base, _RPS)],
                      my_out.at[pl.ds(base, _RPS)])

    @pl.when(sid == _NS - 1)
    def _():
      pltpu.sync_copy(acc_sh.at[pl.ds(base, _RPS + 16)],
                      my_out.at[pl.ds(base, _RPS + 16)])

  return k(h, pk)


_BLK = 1000


def _mlp_body(h_ref, p_ref, w1_ref, b1_ref, w2_ref, b2_ref, o_ref, *, relu_out):
  z = h_ref[...] + p_ref[0] + p_ref[1]
  z1 = lax.dot_general(z, w1_ref[...], (((1,), (1,)), ((), ())),
                       preferred_element_type=jnp.float32) + b1_ref[...]
  z1 = jnp.maximum(z1, 0.0)
  z2 = lax.dot_general(z1, w2_ref[...], (((1,), (1,)), ((), ())),
                       preferred_element_type=jnp.float32) + b2_ref[...]
  if relu_out:
    z2 = jnp.maximum(z2, 0.0)
  o_ref[...] = z2


def _gin_mlp(h, agg, W1, b1, W2, b2, relu_out):
  grid = _N // _BLK
  return pl.pallas_call(
      functools.partial(_mlp_body, relu_out=relu_out),
      grid=(grid,),
      in_specs=[
          pl.BlockSpec((_BLK, _D), lambda i: (i, 0)),
          pl.BlockSpec((_NC, _BLK, _D), lambda i: (0, i, 0)),
          pl.BlockSpec((_D, _D), lambda i: (0, 0)),
          pl.BlockSpec((1, _D), lambda i: (0, 0)),
          pl.BlockSpec((_D, _D), lambda i: (0, 0)),
          pl.BlockSpec((1, _D), lambda i: (0, 0)),
      ],
      out_specs=pl.BlockSpec((_BLK, _D), lambda i: (i, 0)),
      out_shape=jax.ShapeDtypeStruct((_N, _D), jnp.float32),
  )(h, agg, W1, b1.reshape(1, _D), W2, b2.reshape(1, _D))


def _pool_body(batch_ref, h_ref, wl_ref, bl_ref, o_ref, acc_ref, cnt_ref):
  i = pl.program_id(0)

  @pl.when(i == 0)
  def _():
    acc_ref[...] = jnp.zeros_like(acc_ref)
    cnt_ref[...] = jnp.zeros_like(cnt_ref)

  b = batch_ref[0]
  gid = lax.broadcasted_iota(jnp.int32, (_G, _BLK), 0)
  mask = jnp.where(gid == b, 1.0, 0.0).astype(jnp.float32)
  acc_ref[...] += lax.dot_general(mask, h_ref[...], (((1,), (0,)), ((), ())),
                                  preferred_element_type=jnp.float32)
  cnt_ref[...] += jnp.sum(mask, axis=1, keepdims=True)

  @pl.when(i == pl.num_programs(0) - 1)
  def _():
    pooled = acc_ref[...] / jnp.maximum(cnt_ref[...], 1.0)
    logits = lax.dot_general(pooled, wl_ref[...], (((1,), (1,)), ((), ())),
                             preferred_element_type=jnp.float32) + bl_ref[...]
    m = jnp.max(logits, axis=1, keepdims=True)
    ls = m + jnp.log(jnp.sum(jnp.exp(logits - m), axis=1, keepdims=True))
    o_ref[...] = logits - ls


def _pool_classify(h, batch3, Wl, bl):
  grid = _N // _BLK
  return pl.pallas_call(
      _pool_body,
      grid=(grid,),
      in_specs=[
          pl.BlockSpec((1, 1, _BLK), lambda i: (i, 0, 0)),
          pl.BlockSpec((_BLK, _D), lambda i: (i, 0)),
          pl.BlockSpec((_C, _D), lambda i: (0, 0)),
          pl.BlockSpec((1, _C), lambda i: (0, 0)),
      ],
      out_specs=pl.BlockSpec((_G, _C), lambda i: (0, 0)),
      out_shape=jax.ShapeDtypeStruct((_G, _C), jnp.float32),
      scratch_shapes=[
          pltpu.VMEM((_G, _D), jnp.float32),
          pltpu.VMEM((_G, 1), jnp.float32),
      ],
  )(batch3, h, Wl, bl.reshape(1, _C))


def kernel(x, edge_index, edge_attr, batch,
           W1_0, b1_0, W2_0, b2_0,
           W1_1, b1_1, W2_1, b2_1,
           W1_2, b1_2, W2_2, b2_2,
           Wl, bl):
  src = edge_index[0].astype(jnp.int32)
  dst = edge_index[1].astype(jnp.int32)
  padv = jnp.zeros((_NC, _NS, _PAD), jnp.int32)
  srcw = jnp.concatenate([src.reshape(_NC, _NS, _EPS), padv], axis=-1)
  dstw = jnp.concatenate([dst.reshape(_NC, _NS, _EPS), padv + _N], axis=-1)
  srcw = srcw.reshape(_NC, _NS, _NGRP, _IB, _CHUNK)
  dstw = dstw.reshape(_NC, _NS, _NGRP, _IB, _CHUNK)
  pk = jnp.stack([srcw, dstw], axis=4)
  batch3 = batch.astype(jnp.int32).reshape(_N // _BLK, 1, _BLK)

  h = x
  params = [(W1_0, b1_0, W2_0, b2_0, True),
            (W1_1, b1_1, W2_1, b2_1, True),
            (W1_2, b1_2, W2_2, b2_2, False)]
  for (W1, b1, W2, b2, relu_out) in params:
    agg = _seg_sum(h, pk)
    h = _gin_mlp(h, agg, W1, b1, W2, b2, relu_out)

  return _pool_classify(h, batch3, Wl, bl)


# speedup vs baseline: 1.0002x; 1.0002x over previous
import functools

import jax
import jax.numpy as jnp
from jax import lax
from jax.experimental import pallas as pl
from jax.experimental.pallas import tpu as pltpu
from jax.experimental.pallas import tpu_sc as plsc

_N = 10000
_E = 320000
_D = 128
_G = 64
_C = 10

_NC = 2
_NS = 16
_EPC = _E // _NC
_EPS = _EPC // _NS
_CHUNK = 64
_EPSP = 10240
_PAD = _EPSP - _EPS
_IB = 10
_NGRP = _EPSP // (_CHUNK * _IB)
_AROWS = _N + 16
_RPS = 624
_ZROWS = 16


def _seg_sum(h, pk):
  mesh = plsc.VectorSubcoreMesh(core_axis_name="c", subcore_axis_name="s")

  @functools.partial(
      pl.kernel,
      out_type=jax.ShapeDtypeStruct((_NC, _N, _D), jnp.float32),
      mesh=mesh,
      scratch_types=[
          pltpu.VMEM((_IB, 2, _CHUNK), jnp.int32),
          pltpu.VMEM((_CHUNK, _D), jnp.float32),
          pltpu.VMEM((_CHUNK, _D), jnp.float32),
          pltpu.VMEM((_ZROWS, _D), jnp.float32),
          pltpu.VMEM_SHARED((_AROWS, _D), jnp.float32),
          pltpu.SemaphoreType.DMA,
          pltpu.SemaphoreType.DMA,
      ],
  )
  def k(h_hbm, pk_hbm, out_hbm,
        ib, rows0, rows1, zero_v, acc_sh, sem0, sem1):
    cid = lax.axis_index("c")
    sid = lax.axis_index("s")
    my_pk = pk_hbm.at[cid].at[sid]

    @pl.loop(0, _ZROWS)
    def _(r):
      @pl.loop(0, _D, step=16)
      def _(c0):
        zero_v[r, pl.ds(c0, 16)] = jnp.zeros((16,), jnp.float32)

    base = sid * _RPS

    @pl.loop(0, _RPS // _ZROWS)
    def _(i):
      pltpu.sync_copy(zero_v, acc_sh.at[pl.ds(base + i * _ZROWS, _ZROWS)])

    @pl.when(sid == _NS - 1)
    def _():
      pltpu.sync_copy(zero_v, acc_sh.at[pl.ds(base + _RPS, _ZROWS)])

    plsc.subcore_barrier()

    @pl.loop(0, _NGRP)
    def _(g):
      pltpu.sync_copy(my_pk.at[g], ib)
      pltpu.async_copy(h_hbm.at[ib.at[0].at[0]], rows0, sem0)

      @pl.loop(0, _IB, step=2)
      def _(t):
        pltpu.async_copy(h_hbm.at[ib.at[t + 1].at[0]], rows1, sem1)
        pltpu.make_async_copy(h_hbm.at[ib.at[t].at[0]], rows0, sem0).wait()
        pltpu.sync_copy(rows0, acc_sh.at[ib.at[t].at[1]], add=True)

        @pl.when(t < _IB - 2)
        def _():
          pltpu.async_copy(h_hbm.at[ib.at[t + 2].at[0]], rows0, sem0)

        pltpu.make_async_copy(h_hbm.at[ib.at[t + 1].at[0]], rows1, sem1).wait()
        pltpu.sync_copy(rows1, acc_sh.at[ib.at[t + 1].at[1]], add=True)

    plsc.subcore_barrier()

    my_out = out_hbm.at[cid]

    @pl.when(sid < _NS - 1)
    def _():
      pltpu.sync_copy(acc_sh.at[pl.ds(base, _RPS)],
                      my_out.at[pl.ds(base, _RPS)])

    @pl.when(sid == _NS - 1)
    def _():
      pltpu.sync_copy(acc_sh.at[pl.ds(base, _RPS + 16)],
                      my_out.at[pl.ds(base, _RPS + 16)])

  return k(h, pk)


_BLK = 1000


def _mlp_body(h_ref, p_ref, w1_ref, b1_ref, w2_ref, b2_ref, o_ref, *, relu_out):
  z = h_ref[...] + p_ref[0] + p_ref[1]
  z1 = lax.dot_general(z, w1_ref[...], (((1,), (1,)), ((), ())),
                       preferred_element_type=jnp.float32) + b1_ref[...]
  z1 = jnp.maximum(z1, 0.0)
  z2 = lax.dot_general(z1, w2_ref[...], (((1,), (1,)), ((), ())),
                       preferred_element_type=jnp.float32) + b2_ref[...]
  if relu_out:
    z2 = jnp.maximum(z2, 0.0)
  o_ref[...] = z2


def _gin_mlp(h, agg, W1, b1, W2, b2, relu_out):
  grid = _N // _BLK
  return pl.pallas_call(
      functools.partial(_mlp_body, relu_out=relu_out),
      grid=(grid,),
      in_specs=[
          pl.BlockSpec((_BLK, _D), lambda i: (i, 0)),
          pl.BlockSpec((_NC, _BLK, _D), lambda i: (0, i, 0)),
          pl.BlockSpec((_D, _D), lambda i: (0, 0)),
          pl.BlockSpec((1, _D), lambda i: (0, 0)),
          pl.BlockSpec((_D, _D), lambda i: (0, 0)),
          pl.BlockSpec((1, _D), lambda i: (0, 0)),
      ],
      out_specs=pl.BlockSpec((_BLK, _D), lambda i: (i, 0)),
      out_shape=jax.ShapeDtypeStruct((_N, _D), jnp.float32),
  )(h, agg, W1, b1.reshape(1, _D), W2, b2.reshape(1, _D))


def _pool_body(batch_ref, h_ref, wl_ref, bl_ref, o_ref, acc_ref, cnt_ref):
  i = pl.program_id(0)

  @pl.when(i == 0)
  def _():
    acc_ref[...] = jnp.zeros_like(acc_ref)
    cnt_ref[...] = jnp.zeros_like(cnt_ref)

  b = batch_ref[0]
  gid = lax.broadcasted_iota(jnp.int32, (_G, _BLK), 0)
  mask = jnp.where(gid == b, 1.0, 0.0).astype(jnp.float32)
  acc_ref[...] += lax.dot_general(mask, h_ref[...], (((1,), (0,)), ((), ())),
                                  preferred_element_type=jnp.float32)
  cnt_ref[...] += jnp.sum(mask, axis=1, keepdims=True)

  @pl.when(i == pl.num_programs(0) - 1)
  def _():
    pooled = acc_ref[...] / jnp.maximum(cnt_ref[...], 1.0)
    logits = lax.dot_general(pooled, wl_ref[...], (((1,), (1,)), ((), ())),
                             preferred_element_type=jnp.float32) + bl_ref[...]
    m = jnp.max(logits, axis=1, keepdims=True)
    ls = m + jnp.log(jnp.sum(jnp.exp(logits - m), axis=1, keepdims=True))
    o_ref[...] = logits - ls


def _pool_classify(h, batch3, Wl, bl):
  grid = _N // _BLK
  return pl.pallas_call(
      _pool_body,
      grid=(grid,),
      in_specs=[
          pl.BlockSpec((1, 1, _BLK), lambda i: (i, 0, 0)),
          pl.BlockSpec((_BLK, _D), lambda i: (i, 0)),
          pl.BlockSpec((_C, _D), lambda i: (0, 0)),
          pl.BlockSpec((1, _C), lambda i: (0, 0)),
      ],
      out_specs=pl.BlockSpec((_G, _C), lambda i: (0, 0)),
      out_shape=jax.ShapeDtypeStruct((_G, _C), jnp.float32),
      scratch_shapes=[
          pltpu.VMEM((_G, _D), jnp.float32),
          pltpu.VMEM((_G, 1), jnp.float32),
      ],
  )(batch3, h, Wl, bl.reshape(1, _C))


def kernel(x, edge_index, edge_attr, batch,
           W1_0, b1_0, W2_0, b2_0,
           W1_1, b1_1, W2_1, b2_1,
           W1_2, b1_2, W2_2, b2_2,
           Wl, bl):
  src = edge_index[0].astype(jnp.int32)
  dst = edge_index[1].astype(jnp.int32)
  padv = jnp.zeros((_NC, _NS, _PAD), jnp.int32)
  padt = _N + (lax.iota(jnp.int32, _PAD) & 15)
  padd = jnp.broadcast_to(padt, (_NC, _NS, _PAD))
  srcw = jnp.concatenate([src.reshape(_NC, _NS, _EPS), padv], axis=-1)
  dstw = jnp.concatenate([dst.reshape(_NC, _NS, _EPS), padd], axis=-1)
  srcw = srcw.reshape(_NC, _NS, _NGRP, _IB, _CHUNK)
  dstw = dstw.reshape(_NC, _NS, _NGRP, _IB, _CHUNK)
  pk = jnp.stack([srcw, dstw], axis=4)
  batch3 = batch.astype(jnp.int32).reshape(_N // _BLK, 1, _BLK)

  h = x
  params = [(W1_0, b1_0, W2_0, b2_0, True),
            (W1_1, b1_1, W2_1, b2_1, True),
            (W1_2, b1_2, W2_2, b2_2, False)]
  for (W1, b1, W2, b2, relu_out) in params:
    agg = _seg_sum(h, pk)
    h = _gin_mlp(h, agg, W1, b1, W2, b2, relu_out)

  return _pool_classify(h, batch3, Wl, bl)


# speedup vs baseline: 1.0394x; 1.0392x over previous
import functools

import jax
import jax.numpy as jnp
from jax import lax
from jax.experimental import pallas as pl
from jax.experimental.pallas import tpu as pltpu
from jax.experimental.pallas import tpu_sc as plsc

_N = 10000
_E = 320000
_D = 128
_G = 64
_C = 10

_NC = 2
_NS = 16
_EPC = _E // _NC
_EPS = _EPC // _NS
_CHUNK = 80
_EPSP = 10240
_PAD = _EPSP - _EPS
_IB = 16
_NGRP = _EPSP // (_CHUNK * _IB)
_AROWS = _N + 16
_RPS = 624
_ZROWS = 16


def _seg_sum(h, pk):
  mesh = plsc.VectorSubcoreMesh(core_axis_name="c", subcore_axis_name="s")

  @functools.partial(
      pl.kernel,
      out_type=jax.ShapeDtypeStruct((_NC, _N, _D), jnp.float32),
      mesh=mesh,
      scratch_types=[
          pltpu.VMEM((_IB, 2, _CHUNK), jnp.int32),
          pltpu.VMEM((_CHUNK, _D), jnp.float32),
          pltpu.VMEM((_CHUNK, _D), jnp.float32),
          pltpu.VMEM((_ZROWS, _D), jnp.float32),
          pltpu.VMEM_SHARED((_AROWS, _D), jnp.float32),
          pltpu.SemaphoreType.DMA,
          pltpu.SemaphoreType.DMA,
      ],
  )
  def k(h_hbm, pk_hbm, out_hbm,
        ib, rows0, rows1, zero_v, acc_sh, sem0, sem1):
    cid = lax.axis_index("c")
    sid = lax.axis_index("s")
    my_pk = pk_hbm.at[cid].at[sid]

    @pl.loop(0, _ZROWS)
    def _(r):
      @pl.loop(0, _D, step=16)
      def _(c0):
        zero_v[r, pl.ds(c0, 16)] = jnp.zeros((16,), jnp.float32)

    base = sid * _RPS

    @pl.loop(0, _RPS // _ZROWS)
    def _(i):
      pltpu.sync_copy(zero_v, acc_sh.at[pl.ds(base + i * _ZROWS, _ZROWS)])

    @pl.when(sid == _NS - 1)
    def _():
      pltpu.sync_copy(zero_v, acc_sh.at[pl.ds(base + _RPS, _ZROWS)])

    plsc.subcore_barrier()

    @pl.loop(0, _NGRP)
    def _(g):
      pltpu.sync_copy(my_pk.at[g], ib)
      pltpu.async_copy(h_hbm.at[ib.at[0].at[0]], rows0, sem0)

      @pl.loop(0, _IB, step=2)
      def _(t):
        pltpu.async_copy(h_hbm.at[ib.at[t + 1].at[0]], rows1, sem1)
        pltpu.make_async_copy(h_hbm.at[ib.at[t].at[0]], rows0, sem0).wait()
        pltpu.sync_copy(rows0, acc_sh.at[ib.at[t].at[1]], add=True)

        @pl.when(t < _IB - 2)
        def _():
          pltpu.async_copy(h_hbm.at[ib.at[t + 2].at[0]], rows0, sem0)

        pltpu.make_async_copy(h_hbm.at[ib.at[t + 1].at[0]], rows1, sem1).wait()
        pltpu.sync_copy(rows1, acc_sh.at[ib.at[t + 1].at[1]], add=True)

    plsc.subcore_barrier()

    my_out = out_hbm.at[cid]

    @pl.when(sid < _NS - 1)
    def _():
      pltpu.sync_copy(acc_sh.at[pl.ds(base, _RPS)],
                      my_out.at[pl.ds(base, _RPS)])

    @pl.when(sid == _NS - 1)
    def _():
      pltpu.sync_copy(acc_sh.at[pl.ds(base, _RPS + 16)],
                      my_out.at[pl.ds(base, _RPS + 16)])

  return k(h, pk)


_BLK = 1000


def _mlp_body(h_ref, p_ref, w1_ref, b1_ref, w2_ref, b2_ref, o_ref, *, relu_out):
  z = h_ref[...] + p_ref[0] + p_ref[1]
  z1 = lax.dot_general(z, w1_ref[...], (((1,), (1,)), ((), ())),
                       preferred_element_type=jnp.float32) + b1_ref[...]
  z1 = jnp.maximum(z1, 0.0)
  z2 = lax.dot_general(z1, w2_ref[...], (((1,), (1,)), ((), ())),
                       preferred_element_type=jnp.float32) + b2_ref[...]
  if relu_out:
    z2 = jnp.maximum(z2, 0.0)
  o_ref[...] = z2


def _gin_mlp(h, agg, W1, b1, W2, b2, relu_out):
  grid = _N // _BLK
  return pl.pallas_call(
      functools.partial(_mlp_body, relu_out=relu_out),
      grid=(grid,),
      in_specs=[
          pl.BlockSpec((_BLK, _D), lambda i: (i, 0)),
          pl.BlockSpec((_NC, _BLK, _D), lambda i: (0, i, 0)),
          pl.BlockSpec((_D, _D), lambda i: (0, 0)),
          pl.BlockSpec((1, _D), lambda i: (0, 0)),
          pl.BlockSpec((_D, _D), lambda i: (0, 0)),
          pl.BlockSpec((1, _D), lambda i: (0, 0)),
      ],
      out_specs=pl.BlockSpec((_BLK, _D), lambda i: (i, 0)),
      out_shape=jax.ShapeDtypeStruct((_N, _D), jnp.float32),
  )(h, agg, W1, b1.reshape(1, _D), W2, b2.reshape(1, _D))


def _pool_body(batch_ref, h_ref, wl_ref, bl_ref, o_ref, acc_ref, cnt_ref):
  i = pl.program_id(0)

  @pl.when(i == 0)
  def _():
    acc_ref[...] = jnp.zeros_like(acc_ref)
    cnt_ref[...] = jnp.zeros_like(cnt_ref)

  b = batch_ref[0]
  gid = lax.broadcasted_iota(jnp.int32, (_G, _BLK), 0)
  mask = jnp.where(gid == b, 1.0, 0.0).astype(jnp.float32)
  acc_ref[...] += lax.dot_general(mask, h_ref[...], (((1,), (0,)), ((), ())),
                                  preferred_element_type=jnp.float32)
  cnt_ref[...] += jnp.sum(mask, axis=1, keepdims=True)

  @pl.when(i == pl.num_programs(0) - 1)
  def _():
    pooled = acc_ref[...] / jnp.maximum(cnt_ref[...], 1.0)
    logits = lax.dot_general(pooled, wl_ref[...], (((1,), (1,)), ((), ())),
                             preferred_element_type=jnp.float32) + bl_ref[...]
    m = jnp.max(logits, axis=1, keepdims=True)
    ls = m + jnp.log(jnp.sum(jnp.exp(logits - m), axis=1, keepdims=True))
    o_ref[...] = logits - ls


def _pool_classify(h, batch3, Wl, bl):
  grid = _N // _BLK
  return pl.pallas_call(
      _pool_body,
      grid=(grid,),
      in_specs=[
          pl.BlockSpec((1, 1, _BLK), lambda i: (i, 0, 0)),
          pl.BlockSpec((_BLK, _D), lambda i: (i, 0)),
          pl.BlockSpec((_C, _D), lambda i: (0, 0)),
          pl.BlockSpec((1, _C), lambda i: (0, 0)),
      ],
      out_specs=pl.BlockSpec((_G, _C), lambda i: (0, 0)),
      out_shape=jax.ShapeDtypeStruct((_G, _C), jnp.float32),
      scratch_shapes=[
          pltpu.VMEM((_G, _D), jnp.float32),
          pltpu.VMEM((_G, 1), jnp.float32),
      ],
  )(batch3, h, Wl, bl.reshape(1, _C))


def kernel(x, edge_index, edge_attr, batch,
           W1_0, b1_0, W2_0, b2_0,
           W1_1, b1_1, W2_1, b2_1,
           W1_2, b1_2, W2_2, b2_2,
           Wl, bl):
  src = edge_index[0].astype(jnp.int32)
  dst = edge_index[1].astype(jnp.int32)
  padv = jnp.zeros((_NC, _NS, _PAD), jnp.int32)
  padt = _N + (lax.iota(jnp.int32, _PAD) & 15)
  padd = jnp.broadcast_to(padt, (_NC, _NS, _PAD))
  srcw = jnp.concatenate([src.reshape(_NC, _NS, _EPS), padv], axis=-1)
  dstw = jnp.concatenate([dst.reshape(_NC, _NS, _EPS), padd], axis=-1)
  srcw = srcw.reshape(_NC, _NS, _NGRP, _IB, _CHUNK)
  dstw = dstw.reshape(_NC, _NS, _NGRP, _IB, _CHUNK)
  pk = jnp.stack([srcw, dstw], axis=4)
  batch3 = batch.astype(jnp.int32).reshape(_N // _BLK, 1, _BLK)

  h = x
  params = [(W1_0, b1_0, W2_0, b2_0, True),
            (W1_1, b1_1, W2_1, b2_1, True),
            (W1_2, b1_2, W2_2, b2_2, False)]
  for (W1, b1, W2, b2, relu_out) in params:
    agg = _seg_sum(h, pk)
    h = _gin_mlp(h, agg, W1, b1, W2, b2, relu_out)

  return _pool_classify(h, batch3, Wl, bl)


# speedup vs baseline: 2.8594x; 2.7511x over previous
import functools

import jax
import jax.numpy as jnp
from jax import lax
from jax.experimental import pallas as pl
from jax.experimental.pallas import tpu as pltpu
from jax.experimental.pallas import tpu_sc as plsc

_N = 10000
_E = 320000
_D = 128
_G = 64
_C = 10

_NC = 2
_NS = 16
_EPC = _E // _NC
_EPS = _EPC // _NS
_CHUNK = 80
_EPSP = 10240
_PAD = _EPSP - _EPS
_IB = 16
_NGRP = _EPSP // (_CHUNK * _IB)
_AROWS = _N + 16
_RPS = 624
_ZROWS = 16


def _seg_sum(h, pk):
  mesh = plsc.VectorSubcoreMesh(core_axis_name="c", subcore_axis_name="s")

  @functools.partial(
      pl.kernel,
      out_type=jax.ShapeDtypeStruct((_NC, _N, _D), jnp.float32),
      mesh=mesh,
      scratch_types=[
          pltpu.VMEM((_IB, 2, _CHUNK), jnp.int32),
          pltpu.VMEM((_CHUNK, _D), jnp.float32),
          pltpu.VMEM((_CHUNK, _D), jnp.float32),
          pltpu.VMEM((_ZROWS, _D), jnp.float32),
          pltpu.VMEM_SHARED((_AROWS, _D), jnp.float32),
          pltpu.SemaphoreType.DMA,
          pltpu.SemaphoreType.DMA,
      ],
  )
  def k(h_hbm, pk_hbm, out_hbm,
        ib, rows0, rows1, zero_v, acc_sh, sem0, sem1):
    cid = lax.axis_index("c")
    sid = lax.axis_index("s")
    my_pk = pk_hbm.at[cid].at[sid]

    @pl.loop(0, _ZROWS)
    def _(r):
      @pl.loop(0, _D, step=16)
      def _(c0):
        zero_v[r, pl.ds(c0, 16)] = jnp.zeros((16,), jnp.float32)

    base = sid * _RPS

    @pl.loop(0, _RPS // _ZROWS)
    def _(i):
      pltpu.sync_copy(zero_v, acc_sh.at[pl.ds(base + i * _ZROWS, _ZROWS)])

    @pl.when(sid == _NS - 1)
    def _():
      pltpu.sync_copy(zero_v, acc_sh.at[pl.ds(base + _RPS, _ZROWS)])

    plsc.subcore_barrier()

    @pl.loop(0, _NGRP)
    def _(g):
      pltpu.sync_copy(my_pk.at[g], ib)
      pltpu.async_copy(h_hbm.at[ib.at[0].at[0]], rows0, sem0)

      @pl.loop(0, _IB, step=2)
      def _(t):
        pltpu.async_copy(h_hbm.at[ib.at[t + 1].at[0]], rows1, sem1)
        pltpu.make_async_copy(h_hbm.at[ib.at[t].at[0]], rows0, sem0).wait()
        pltpu.sync_copy(rows0, acc_sh.at[ib.at[t].at[1]], add=True)

        @pl.when(t < _IB - 2)
        def _():
          pltpu.async_copy(h_hbm.at[ib.at[t + 2].at[0]], rows0, sem0)

        pltpu.make_async_copy(h_hbm.at[ib.at[t + 1].at[0]], rows1, sem1).wait()
        pltpu.sync_copy(rows1, acc_sh.at[ib.at[t + 1].at[1]], add=True)

    plsc.subcore_barrier()

    my_out = out_hbm.at[cid]

    @pl.when(sid < _NS - 1)
    def _():
      pltpu.sync_copy(acc_sh.at[pl.ds(base, _RPS)],
                      my_out.at[pl.ds(base, _RPS)])

    @pl.when(sid == _NS - 1)
    def _():
      pltpu.sync_copy(acc_sh.at[pl.ds(base, _RPS + 16)],
                      my_out.at[pl.ds(base, _RPS + 16)])

  return k(h, pk)


_BLK = 1000


def _mlp_body(h_ref, p_ref, w1_ref, b1_ref, w2_ref, b2_ref, o_ref, *, relu_out):
  z = h_ref[...] + p_ref[0] + p_ref[1]
  z1 = lax.dot_general(z, w1_ref[...], (((1,), (1,)), ((), ())),
                       preferred_element_type=jnp.float32) + b1_ref[...]
  z1 = jnp.maximum(z1, 0.0)
  z2 = lax.dot_general(z1, w2_ref[...], (((1,), (1,)), ((), ())),
                       preferred_element_type=jnp.float32) + b2_ref[...]
  if relu_out:
    z2 = jnp.maximum(z2, 0.0)
  o_ref[...] = z2


def _gin_mlp(h, agg, W1, b1, W2, b2, relu_out):
  grid = _N // _BLK
  return pl.pallas_call(
      functools.partial(_mlp_body, relu_out=relu_out),
      grid=(grid,),
      in_specs=[
          pl.BlockSpec((_BLK, _D), lambda i: (i, 0)),
          pl.BlockSpec((_NC, _BLK, _D), lambda i: (0, i, 0)),
          pl.BlockSpec((_D, _D), lambda i: (0, 0)),
          pl.BlockSpec((1, _D), lambda i: (0, 0)),
          pl.BlockSpec((_D, _D), lambda i: (0, 0)),
          pl.BlockSpec((1, _D), lambda i: (0, 0)),
      ],
      out_specs=pl.BlockSpec((_BLK, _D), lambda i: (i, 0)),
      out_shape=jax.ShapeDtypeStruct((_N, _D), jnp.float32),
  )(h, agg, W1, b1.reshape(1, _D), W2, b2.reshape(1, _D))


def _pool_body(batch_ref, h_ref, wl_ref, bl_ref, o_ref, acc_ref, cnt_ref):
  i = pl.program_id(0)

  @pl.when(i == 0)
  def _():
    acc_ref[...] = jnp.zeros_like(acc_ref)
    cnt_ref[...] = jnp.zeros_like(cnt_ref)

  b = batch_ref[0]
  gid = lax.broadcasted_iota(jnp.int32, (_G, _BLK), 0)
  mask = jnp.where(gid == b, 1.0, 0.0).astype(jnp.float32)
  acc_ref[...] += lax.dot_general(mask, h_ref[...], (((1,), (0,)), ((), ())),
                                  preferred_element_type=jnp.float32)
  cnt_ref[...] += jnp.sum(mask, axis=1, keepdims=True)

  @pl.when(i == pl.num_programs(0) - 1)
  def _():
    pooled = acc_ref[...] / jnp.maximum(cnt_ref[...], 1.0)
    logits = lax.dot_general(pooled, wl_ref[...], (((1,), (1,)), ((), ())),
                             preferred_element_type=jnp.float32) + bl_ref[...]
    m = jnp.max(logits, axis=1, keepdims=True)
    ls = m + jnp.log(jnp.sum(jnp.exp(logits - m), axis=1, keepdims=True))
    o_ref[...] = logits - ls


def _pool_classify(h, batch3, Wl, bl):
  grid = _N // _BLK
  return pl.pallas_call(
      _pool_body,
      grid=(grid,),
      in_specs=[
          pl.BlockSpec((1, 1, _BLK), lambda i: (i, 0, 0)),
          pl.BlockSpec((_BLK, _D), lambda i: (i, 0)),
          pl.BlockSpec((_C, _D), lambda i: (0, 0)),
          pl.BlockSpec((1, _C), lambda i: (0, 0)),
      ],
      out_specs=pl.BlockSpec((_G, _C), lambda i: (0, 0)),
      out_shape=jax.ShapeDtypeStruct((_G, _C), jnp.float32),
      scratch_shapes=[
          pltpu.VMEM((_G, _D), jnp.float32),
          pltpu.VMEM((_G, 1), jnp.float32),
      ],
  )(batch3, h, Wl, bl.reshape(1, _C))


def kernel(x, edge_index, edge_attr, batch,
           W1_0, b1_0, W2_0, b2_0,
           W1_1, b1_1, W2_1, b2_1,
           W1_2, b1_2, W2_2, b2_2,
           Wl, bl):
  src = edge_index[0].astype(jnp.int32)
  dst = edge_index[1].astype(jnp.int32)
  pads = jnp.broadcast_to(lax.iota(jnp.int32, _PAD),
                          (_NC, _NS, _PAD))
  padt = _N + (lax.iota(jnp.int32, _PAD) & 15)
  padd = jnp.broadcast_to(padt, (_NC, _NS, _PAD))
  srcw = jnp.concatenate([src.reshape(_NC, _NS, _EPS), pads], axis=-1)
  dstw = jnp.concatenate([dst.reshape(_NC, _NS, _EPS), padd], axis=-1)
  srcw = srcw.reshape(_NC, _NS, _NGRP, _IB, _CHUNK)
  dstw = dstw.reshape(_NC, _NS, _NGRP, _IB, _CHUNK)
  pk = jnp.stack([srcw, dstw], axis=4)
  batch3 = batch.astype(jnp.int32).reshape(_N // _BLK, 1, _BLK)

  h = x
  params = [(W1_0, b1_0, W2_0, b2_0, True),
            (W1_1, b1_1, W2_1, b2_1, True),
            (W1_2, b1_2, W2_2, b2_2, False)]
  for (W1, b1, W2, b2, relu_out) in params:
    agg = _seg_sum(h, pk)
    h = _gin_mlp(h, agg, W1, b1, W2, b2, relu_out)

  return _pool_classify(h, batch3, Wl, bl)


# speedup vs baseline: 3.2213x; 1.1266x over previous
import functools

import jax
import jax.numpy as jnp
from jax import lax
from jax.experimental import pallas as pl
from jax.experimental.pallas import tpu as pltpu
from jax.experimental.pallas import tpu_sc as plsc

_N = 10000
_E = 320000
_D = 128
_G = 64
_C = 10

_NC = 2
_NS = 16
_EPC = _E // _NC
_EPS = _EPC // _NS
_CHUNK = 128
_EPSP = 10240
_PAD = _EPSP - _EPS
_IB = 16
_NGRP = _EPSP // (_CHUNK * _IB)
_AROWS = _N + 16
_RPS = 624
_ZROWS = 16


def _seg_sum(h, pk):
  mesh = plsc.VectorSubcoreMesh(core_axis_name="c", subcore_axis_name="s")

  @functools.partial(
      pl.kernel,
      out_type=jax.ShapeDtypeStruct((_NC, _N, _D), jnp.float32),
      mesh=mesh,
      scratch_types=[
          pltpu.VMEM((_IB, 2, _CHUNK), jnp.int32),
          pltpu.VMEM((_CHUNK, _D), jnp.float32),
          pltpu.VMEM((_CHUNK, _D), jnp.float32),
          pltpu.VMEM((_ZROWS, _D), jnp.float32),
          pltpu.VMEM_SHARED((_AROWS, _D), jnp.float32),
          pltpu.SemaphoreType.DMA,
          pltpu.SemaphoreType.DMA,
      ],
  )
  def k(h_hbm, pk_hbm, out_hbm,
        ib, rows0, rows1, zero_v, acc_sh, sem0, sem1):
    cid = lax.axis_index("c")
    sid = lax.axis_index("s")
    my_pk = pk_hbm.at[cid].at[sid]

    @pl.loop(0, _ZROWS)
    def _(r):
      @pl.loop(0, _D, step=16)
      def _(c0):
        zero_v[r, pl.ds(c0, 16)] = jnp.zeros((16,), jnp.float32)

    base = sid * _RPS

    @pl.loop(0, _RPS // _ZROWS)
    def _(i):
      pltpu.sync_copy(zero_v, acc_sh.at[pl.ds(base + i * _ZROWS, _ZROWS)])

    @pl.when(sid == _NS - 1)
    def _():
      pltpu.sync_copy(zero_v, acc_sh.at[pl.ds(base + _RPS, _ZROWS)])

    plsc.subcore_barrier()

    @pl.loop(0, _NGRP)
    def _(g):
      pltpu.sync_copy(my_pk.at[g], ib)
      pltpu.async_copy(h_hbm.at[ib.at[0].at[0]], rows0, sem0)

      @pl.loop(0, _IB, step=2)
      def _(t):
        pltpu.async_copy(h_hbm.at[ib.at[t + 1].at[0]], rows1, sem1)
        pltpu.make_async_copy(h_hbm.at[ib.at[t].at[0]], rows0, sem0).wait()
        pltpu.sync_copy(rows0, acc_sh.at[ib.at[t].at[1]], add=True)

        @pl.when(t < _IB - 2)
        def _():
          pltpu.async_copy(h_hbm.at[ib.at[t + 2].at[0]], rows0, sem0)

        pltpu.make_async_copy(h_hbm.at[ib.at[t + 1].at[0]], rows1, sem1).wait()
        pltpu.sync_copy(rows1, acc_sh.at[ib.at[t + 1].at[1]], add=True)

    plsc.subcore_barrier()

    my_out = out_hbm.at[cid]

    @pl.when(sid < _NS - 1)
    def _():
      pltpu.sync_copy(acc_sh.at[pl.ds(base, _RPS)],
                      my_out.at[pl.ds(base, _RPS)])

    @pl.when(sid == _NS - 1)
    def _():
      pltpu.sync_copy(acc_sh.at[pl.ds(base, _RPS + 16)],
                      my_out.at[pl.ds(base, _RPS + 16)])

  return k(h, pk)


_BLK = 1000


def _mlp_body(h_ref, p_ref, w1_ref, b1_ref, w2_ref, b2_ref, o_ref, *, relu_out):
  z = h_ref[...] + p_ref[0] + p_ref[1]
  z1 = lax.dot_general(z, w1_ref[...], (((1,), (1,)), ((), ())),
                       preferred_element_type=jnp.float32) + b1_ref[...]
  z1 = jnp.maximum(z1, 0.0)
  z2 = lax.dot_general(z1, w2_ref[...], (((1,), (1,)), ((), ())),
                       preferred_element_type=jnp.float32) + b2_ref[...]
  if relu_out:
    z2 = jnp.maximum(z2, 0.0)
  o_ref[...] = z2


def _gin_mlp(h, agg, W1, b1, W2, b2, relu_out):
  grid = _N // _BLK
  return pl.pallas_call(
      functools.partial(_mlp_body, relu_out=relu_out),
      grid=(grid,),
      in_specs=[
          pl.BlockSpec((_BLK, _D), lambda i: (i, 0)),
          pl.BlockSpec((_NC, _BLK, _D), lambda i: (0, i, 0)),
          pl.BlockSpec((_D, _D), lambda i: (0, 0)),
          pl.BlockSpec((1, _D), lambda i: (0, 0)),
          pl.BlockSpec((_D, _D), lambda i: (0, 0)),
          pl.BlockSpec((1, _D), lambda i: (0, 0)),
      ],
      out_specs=pl.BlockSpec((_BLK, _D), lambda i: (i, 0)),
      out_shape=jax.ShapeDtypeStruct((_N, _D), jnp.float32),
  )(h, agg, W1, b1.reshape(1, _D), W2, b2.reshape(1, _D))


def _pool_body(batch_ref, h_ref, wl_ref, bl_ref, o_ref, acc_ref, cnt_ref):
  i = pl.program_id(0)

  @pl.when(i == 0)
  def _():
    acc_ref[...] = jnp.zeros_like(acc_ref)
    cnt_ref[...] = jnp.zeros_like(cnt_ref)

  b = batch_ref[0]
  gid = lax.broadcasted_iota(jnp.int32, (_G, _BLK), 0)
  mask = jnp.where(gid == b, 1.0, 0.0).astype(jnp.float32)
  acc_ref[...] += lax.dot_general(mask, h_ref[...], (((1,), (0,)), ((), ())),
                                  preferred_element_type=jnp.float32)
  cnt_ref[...] += jnp.sum(mask, axis=1, keepdims=True)

  @pl.when(i == pl.num_programs(0) - 1)
  def _():
    pooled = acc_ref[...] / jnp.maximum(cnt_ref[...], 1.0)
    logits = lax.dot_general(pooled, wl_ref[...], (((1,), (1,)), ((), ())),
                             preferred_element_type=jnp.float32) + bl_ref[...]
    m = jnp.max(logits, axis=1, keepdims=True)
    ls = m + jnp.log(jnp.sum(jnp.exp(logits - m), axis=1, keepdims=True))
    o_ref[...] = logits - ls


def _pool_classify(h, batch3, Wl, bl):
  grid = _N // _BLK
  return pl.pallas_call(
      _pool_body,
      grid=(grid,),
      in_specs=[
          pl.BlockSpec((1, 1, _BLK), lambda i: (i, 0, 0)),
          pl.BlockSpec((_BLK, _D), lambda i: (i, 0)),
          pl.BlockSpec((_C, _D), lambda i: (0, 0)),
          pl.BlockSpec((1, _C), lambda i: (0, 0)),
      ],
      out_specs=pl.BlockSpec((_G, _C), lambda i: (0, 0)),
      out_shape=jax.ShapeDtypeStruct((_G, _C), jnp.float32),
      scratch_shapes=[
          pltpu.VMEM((_G, _D), jnp.float32),
          pltpu.VMEM((_G, 1), jnp.float32),
      ],
  )(batch3, h, Wl, bl.reshape(1, _C))


def kernel(x, edge_index, edge_attr, batch,
           W1_0, b1_0, W2_0, b2_0,
           W1_1, b1_1, W2_1, b2_1,
           W1_2, b1_2, W2_2, b2_2,
           Wl, bl):
  src = edge_index[0].astype(jnp.int32)
  dst = edge_index[1].astype(jnp.int32)
  pads = jnp.broadcast_to(lax.iota(jnp.int32, _PAD),
                          (_NC, _NS, _PAD))
  padt = _N + (lax.iota(jnp.int32, _PAD) & 15)
  padd = jnp.broadcast_to(padt, (_NC, _NS, _PAD))
  srcw = jnp.concatenate([src.reshape(_NC, _NS, _EPS), pads], axis=-1)
  dstw = jnp.concatenate([dst.reshape(_NC, _NS, _EPS), padd], axis=-1)
  srcw = srcw.reshape(_NC, _NS, _NGRP, _IB, _CHUNK)
  dstw = dstw.reshape(_NC, _NS, _NGRP, _IB, _CHUNK)
  pk = jnp.stack([srcw, dstw], axis=4)
  batch3 = batch.astype(jnp.int32).reshape(_N // _BLK, 1, _BLK)

  h = x
  params = [(W1_0, b1_0, W2_0, b2_0, True),
            (W1_1, b1_1, W2_1, b2_1, True),
            (W1_2, b1_2, W2_2, b2_2, False)]
  for (W1, b1, W2, b2, relu_out) in params:
    agg = _seg_sum(h, pk)
    h = _gin_mlp(h, agg, W1, b1, W2, b2, relu_out)

  return _pool_classify(h, batch3, Wl, bl)


# speedup vs baseline: 3.4091x; 1.0583x over previous
import functools

import jax
import jax.numpy as jnp
from jax import lax
from jax.experimental import pallas as pl
from jax.experimental.pallas import tpu as pltpu
from jax.experimental.pallas import tpu_sc as plsc

_N = 10000
_E = 320000
_D = 128
_G = 64
_C = 10

_NC = 2
_NS = 16
_EPC = _E // _NC
_EPS = _EPC // _NS
_CHUNK = 128
_EPSP = 10240
_PAD = _EPSP - _EPS
_IB = 8
_NGRP = _EPSP // (_CHUNK * _IB)
_AROWS = _N + 16
_RPS = 624
_ZROWS = 16


def _seg_sum(h, pk):
  mesh = plsc.VectorSubcoreMesh(core_axis_name="c", subcore_axis_name="s")

  @functools.partial(
      pl.kernel,
      out_type=jax.ShapeDtypeStruct((_NC, _N, _D), jnp.float32),
      mesh=mesh,
      scratch_types=[
          pltpu.VMEM((_IB, 2, _CHUNK), jnp.int32),
          pltpu.VMEM((_IB, 2, _CHUNK), jnp.int32),
          pltpu.VMEM((_CHUNK, _D), jnp.float32),
          pltpu.VMEM((_CHUNK, _D), jnp.float32),
          pltpu.VMEM((_ZROWS, _D), jnp.float32),
          pltpu.VMEM_SHARED((_AROWS, _D), jnp.float32),
          pltpu.SemaphoreType.DMA,
          pltpu.SemaphoreType.DMA,
          pltpu.SemaphoreType.DMA,
      ],
  )
  def k(h_hbm, pk_hbm, out_hbm,
        iba, ibb, rows0, rows1, zero_v, acc_sh, sem0, sem1, semib):
    cid = lax.axis_index("c")
    sid = lax.axis_index("s")
    my_pk = pk_hbm.at[cid].at[sid]

    @pl.loop(0, _ZROWS)
    def _(r):
      @pl.loop(0, _D, step=16)
      def _(c0):
        zero_v[r, pl.ds(c0, 16)] = jnp.zeros((16,), jnp.float32)

    base = sid * _RPS

    @pl.loop(0, _RPS // _ZROWS)
    def _(i):
      pltpu.sync_copy(zero_v, acc_sh.at[pl.ds(base + i * _ZROWS, _ZROWS)])

    @pl.when(sid == _NS - 1)
    def _():
      pltpu.sync_copy(zero_v, acc_sh.at[pl.ds(base + _RPS, _ZROWS)])

    plsc.subcore_barrier()

    pltpu.sync_copy(my_pk.at[0], iba)
    pltpu.async_copy(h_hbm.at[iba.at[0].at[0]], rows0, sem0)

    @pl.loop(0, _NGRP, step=2)
    def _(g):
      for (cur, nxt, off) in ((iba, ibb, 0), (ibb, iba, 1)):
        gg = g + off

        @pl.when(gg + 1 < _NGRP)
        def _():
          pltpu.async_copy(my_pk.at[gg + 1], nxt, semib)

        @pl.loop(0, _IB, step=2)
        def _(t):
          pltpu.async_copy(h_hbm.at[cur.at[t + 1].at[0]], rows1, sem1)
          pltpu.make_async_copy(h_hbm.at[cur.at[t].at[0]], rows0, sem0).wait()
          pltpu.sync_copy(rows0, acc_sh.at[cur.at[t].at[1]], add=True)

          @pl.when(t < _IB - 2)
          def _():
            pltpu.async_copy(h_hbm.at[cur.at[t + 2].at[0]], rows0, sem0)

          @pl.when(jnp.logical_and(t == _IB - 2, gg + 1 < _NGRP))
          def _():
            pltpu.make_async_copy(my_pk.at[gg + 1], nxt, semib).wait()
            pltpu.async_copy(h_hbm.at[nxt.at[0].at[0]], rows0, sem0)

          pltpu.make_async_copy(h_hbm.at[cur.at[t + 1].at[0]], rows1, sem1).wait()
          pltpu.sync_copy(rows1, acc_sh.at[cur.at[t + 1].at[1]], add=True)

    plsc.subcore_barrier()

    my_out = out_hbm.at[cid]

    @pl.when(sid < _NS - 1)
    def _():
      pltpu.sync_copy(acc_sh.at[pl.ds(base, _RPS)],
                      my_out.at[pl.ds(base, _RPS)])

    @pl.when(sid == _NS - 1)
    def _():
      pltpu.sync_copy(acc_sh.at[pl.ds(base, _RPS + 16)],
                      my_out.at[pl.ds(base, _RPS + 16)])

  return k(h, pk)


_BLK = 1000


def _mlp_body(h_ref, p_ref, w1_ref, b1_ref, w2_ref, b2_ref, o_ref, *, relu_out):
  z = h_ref[...] + p_ref[0] + p_ref[1]
  z1 = lax.dot_general(z, w1_ref[...], (((1,), (1,)), ((), ())),
                       preferred_element_type=jnp.float32) + b1_ref[...]
  z1 = jnp.maximum(z1, 0.0)
  z2 = lax.dot_general(z1, w2_ref[...], (((1,), (1,)), ((), ())),
                       preferred_element_type=jnp.float32) + b2_ref[...]
  if relu_out:
    z2 = jnp.maximum(z2, 0.0)
  o_ref[...] = z2


def _gin_mlp(h, agg, W1, b1, W2, b2, relu_out):
  grid = _N // _BLK
  return pl.pallas_call(
      functools.partial(_mlp_body, relu_out=relu_out),
      grid=(grid,),
      in_specs=[
          pl.BlockSpec((_BLK, _D), lambda i: (i, 0)),
          pl.BlockSpec((_NC, _BLK, _D), lambda i: (0, i, 0)),
          pl.BlockSpec((_D, _D), lambda i: (0, 0)),
          pl.BlockSpec((1, _D), lambda i: (0, 0)),
          pl.BlockSpec((_D, _D), lambda i: (0, 0)),
          pl.BlockSpec((1, _D), lambda i: (0, 0)),
      ],
      out_specs=pl.BlockSpec((_BLK, _D), lambda i: (i, 0)),
      out_shape=jax.ShapeDtypeStruct((_N, _D), jnp.float32),
  )(h, agg, W1, b1.reshape(1, _D), W2, b2.reshape(1, _D))


def _pool_body(batch_ref, h_ref, wl_ref, bl_ref, o_ref, acc_ref, cnt_ref):
  i = pl.program_id(0)

  @pl.when(i == 0)
  def _():
    acc_ref[...] = jnp.zeros_like(acc_ref)
    cnt_ref[...] = jnp.zeros_like(cnt_ref)

  b = batch_ref[0]
  gid = lax.broadcasted_iota(jnp.int32, (_G, _BLK), 0)
  mask = jnp.where(gid == b, 1.0, 0.0).astype(jnp.float32)
  acc_ref[...] += lax.dot_general(mask, h_ref[...], (((1,), (0,)), ((), ())),
                                  preferred_element_type=jnp.float32)
  cnt_ref[...] += jnp.sum(mask, axis=1, keepdims=True)

  @pl.when(i == pl.num_programs(0) - 1)
  def _():
    pooled = acc_ref[...] / jnp.maximum(cnt_ref[...], 1.0)
    logits = lax.dot_general(pooled, wl_ref[...], (((1,), (1,)), ((), ())),
                             preferred_element_type=jnp.float32) + bl_ref[...]
    m = jnp.max(logits, axis=1, keepdims=True)
    ls = m + jnp.log(jnp.sum(jnp.exp(logits - m), axis=1, keepdims=True))
    o_ref[...] = logits - ls


def _pool_classify(h, batch3, Wl, bl):
  grid = _N // _BLK
  return pl.pallas_call(
      _pool_body,
      grid=(grid,),
      in_specs=[
          pl.BlockSpec((1, 1, _BLK), lambda i: (i, 0, 0)),
          pl.BlockSpec((_BLK, _D), lambda i: (i, 0)),
          pl.BlockSpec((_C, _D), lambda i: (0, 0)),
          pl.BlockSpec((1, _C), lambda i: (0, 0)),
      ],
      out_specs=pl.BlockSpec((_G, _C), lambda i: (0, 0)),
      out_shape=jax.ShapeDtypeStruct((_G, _C), jnp.float32),
      scratch_shapes=[
          pltpu.VMEM((_G, _D), jnp.float32),
          pltpu.VMEM((_G, 1), jnp.float32),
      ],
  )(batch3, h, Wl, bl.reshape(1, _C))


def kernel(x, edge_index, edge_attr, batch,
           W1_0, b1_0, W2_0, b2_0,
           W1_1, b1_1, W2_1, b2_1,
           W1_2, b1_2, W2_2, b2_2,
           Wl, bl):
  src = edge_index[0].astype(jnp.int32)
  dst = edge_index[1].astype(jnp.int32)
  pads = jnp.broadcast_to(lax.iota(jnp.int32, _PAD),
                          (_NC, _NS, _PAD))
  padt = _N + (lax.iota(jnp.int32, _PAD) & 15)
  padd = jnp.broadcast_to(padt, (_NC, _NS, _PAD))
  srcw = jnp.concatenate([src.reshape(_NC, _NS, _EPS), pads], axis=-1)
  dstw = jnp.concatenate([dst.reshape(_NC, _NS, _EPS), padd], axis=-1)
  srcw = srcw.reshape(_NC, _NS, _NGRP, _IB, _CHUNK)
  dstw = dstw.reshape(_NC, _NS, _NGRP, _IB, _CHUNK)
  pk = jnp.stack([srcw, dstw], axis=4)
  batch3 = batch.astype(jnp.int32).reshape(_N // _BLK, 1, _BLK)

  h = x
  params = [(W1_0, b1_0, W2_0, b2_0, True),
            (W1_1, b1_1, W2_1, b2_1, True),
            (W1_2, b1_2, W2_2, b2_2, False)]
  for (W1, b1, W2, b2, relu_out) in params:
    agg = _seg_sum(h, pk)
    h = _gin_mlp(h, agg, W1, b1, W2, b2, relu_out)

  return _pool_classify(h, batch3, Wl, bl)
